# baseline (speedup 1.0000x reference)
import functools

import jax
import jax.numpy as jnp
from jax import lax
from jax.experimental import pallas as pl
from jax.experimental.pallas import tpu as pltpu
from jax.experimental.pallas import tpu_sc as plsc

N_USERS_K = 1000000
N_ITEMS_K = 1000000
DIM = 64
B = 16384
LAM = 0.001

NC = 2
NS = 16
NW = NC * NS
RPW = B // NW
CHUNK = 128
NCHUNK = RPW // CHUNK


def _sc_body(ue_hbm, ie_hbm, uid_hbm, pid_hbm, nid_hbm,
             diff_hbm, l2_hbm,
             uidx_v, pidx_v, nidx_v, urows_v, prows_v, nrows_v,
             diff_v, l2_v, sem):
    c = lax.axis_index("c")
    s = lax.axis_index("s")
    wid = s * NC + c
    base = wid * RPW

    pltpu.sync_copy(uid_hbm.at[pl.ds(wid * NCHUNK, NCHUNK)], uidx_v)
    pltpu.sync_copy(pid_hbm.at[pl.ds(wid * NCHUNK, NCHUNK)], pidx_v)
    pltpu.sync_copy(nid_hbm.at[pl.ds(wid * NCHUNK, NCHUNK)], nidx_v)

    copies = []
    for j in range(NCHUNK):
        dst = pl.ds(j * CHUNK, CHUNK)
        copies.append(pltpu.async_copy(ue_hbm.at[uidx_v.at[j]],
                                       urows_v.at[dst], sem))
        copies.append(pltpu.async_copy(ie_hbm.at[pidx_v.at[j]],
                                       prows_v.at[dst], sem))
        copies.append(pltpu.async_copy(ie_hbm.at[nidx_v.at[j]],
                                       nrows_v.at[dst], sem))
    for cp in copies:
        cp.wait()

    lane = lax.iota(jnp.int32, 16)
    zero = jnp.zeros((16,), jnp.float32)

    def group(g, carry):
        rows = g * 16 + lane

        def dim_step(d, acc):
            pos, neg, l2 = acc
            col = jnp.full((16,), d, jnp.int32)
            u = plsc.load_gather(urows_v, [rows, col])
            p = plsc.load_gather(prows_v, [rows, col])
            n = plsc.load_gather(nrows_v, [rows, col])
            return (pos + u * p, neg + u * n,
                    l2 + (u * u + (p * p + n * n)))

        pos, neg, l2 = lax.fori_loop(0, DIM, dim_step, (zero, zero, zero))
        diff_v[pl.ds(g * 16, 16)] = neg - pos
        l2_v[pl.ds(g * 16, 16)] = 0.5 * l2
        return carry

    lax.fori_loop(0, RPW // 16, group, 0)

    pltpu.sync_copy(diff_v, diff_hbm.at[pl.ds(base, RPW)])
    pltpu.sync_copy(l2_v, l2_hbm.at[pl.ds(base, RPW)])


def _tc_body(diff_ref, l2_ref, out_ref):
    x = diff_ref[:]
    sp = jnp.maximum(x, 0.0) + jnp.log1p(jnp.exp(-jnp.abs(x)))
    out_ref[0, 0] = jnp.sum(sp) / B + LAM * (jnp.sum(l2_ref[:]) / B)


def kernel(user_embed, item_embed, user_ids, item_pos_ids, item_neg_ids):
    uid = user_ids.astype(jnp.int32).reshape(NW * NCHUNK, CHUNK)
    pid = item_pos_ids.astype(jnp.int32).reshape(NW * NCHUNK, CHUNK)
    nid = item_neg_ids.astype(jnp.int32).reshape(NW * NCHUNK, CHUNK)

    mesh = plsc.VectorSubcoreMesh(core_axis_name="c", subcore_axis_name="s")
    sc = functools.partial(
        pl.kernel,
        mesh=mesh,
        compiler_params=pltpu.CompilerParams(
            use_tc_tiling_on_sc=False, needs_layout_passes=False),
        out_type=[
            jax.ShapeDtypeStruct((B,), jnp.float32),
            jax.ShapeDtypeStruct((B,), jnp.float32),
        ],
        scratch_types=[
            pltpu.VMEM((NCHUNK, CHUNK), jnp.int32),
            pltpu.VMEM((NCHUNK, CHUNK), jnp.int32),
            pltpu.VMEM((NCHUNK, CHUNK), jnp.int32),
            pltpu.VMEM((RPW, DIM), jnp.float32),
            pltpu.VMEM((RPW, DIM), jnp.float32),
            pltpu.VMEM((RPW, DIM), jnp.float32),
            pltpu.VMEM((RPW,), jnp.float32),
            pltpu.VMEM((RPW,), jnp.float32),
            pltpu.SemaphoreType.DMA,
        ],
    )(_sc_body)
    diff, l2row = sc(user_embed, item_embed, uid, pid, nid)

    out = pl.pallas_call(
        _tc_body,
        out_shape=jax.ShapeDtypeStruct((1, 1), jnp.float32),
        out_specs=pl.BlockSpec(memory_space=pltpu.SMEM),
    )(diff.reshape(B // 128, 128), l2row.reshape(B // 128, 128))
    return out[0, 0]
